# baseline (speedup 1.0000x reference)
import jax
import jax.numpy as jnp
from jax import lax
from jax.experimental import pallas as pl
from jax.experimental.pallas import tpu as pltpu
from jax.experimental.pallas import tpu_sc as plsc

B = 16384
D = 128
NC = 2
NS = 16
L = 16
NW = NC * NS
BPW = B // NW
CH = 128
NCH = BPW // CH
NBUF = 3


def _mf_body(user_table, item_table, users_r, items_r, out_hbm,
             uidx, iidx, urows, irows, psum, out_v, sems_u, sems_i):
    wid = lax.axis_index("s") * NC + lax.axis_index("c")

    base = wid * BPW
    pltpu.sync_copy(users_r.at[pl.ds(base, BPW)], uidx)
    pltpu.sync_copy(items_r.at[pl.ds(base, BPW)], iidx)

    def start(c):
        b = c % NBUF
        sl = pl.ds(c * CH, CH)
        cu = pltpu.make_async_copy(user_table.at[uidx.at[sl]], urows.at[b],
                                   sems_u.at[b])
        ci = pltpu.make_async_copy(item_table.at[iidx.at[sl]], irows.at[b],
                                   sems_i.at[b])
        cu.start()
        ci.start()
        return cu, ci

    row_iota = lax.iota(jnp.int32, L)
    pending = [start(c) for c in range(NBUF)]
    for c in range(NCH):
        cur = pending[c % NBUF]
        cur[0].wait()
        cur[1].wait()
        b = c % NBUF
        ub = urows.at[b]
        ib = irows.at[b]

        def gbody(g, _, ub=ub, ib=ib, c=c):
            for j in range(L):
                r = g * L + j
                acc = ub[r, pl.ds(0, L)] * ib[r, pl.ds(0, L)]
                for k in range(1, D // L):
                    sl = pl.ds(k * L, L)
                    acc = acc + ub[r, sl] * ib[r, sl]
                psum[pl.ds(j * (L + 1), L)] = acc
            rows17 = row_iota * (L + 1)
            t = [plsc.load_gather(psum, [rows17 + m]) for m in range(L)]
            while len(t) > 1:
                t = [t[i] + t[i + 1] for i in range(0, len(t), 2)]
            out_v[pl.ds(c * CH + g * L, L)] = t[0]
            return 0

        lax.fori_loop(0, CH // L, gbody, 0)
        if c + NBUF < NCH:
            pending[c % NBUF] = start(c + NBUF)

    pltpu.sync_copy(out_v, out_hbm.at[pl.ds(base, BPW)])


@jax.jit
def _run(users, items, user_table, item_table):
    users_r = users.astype(jnp.int32)
    items_r = items.astype(jnp.int32)
    mesh = plsc.VectorSubcoreMesh(core_axis_name="c", subcore_axis_name="s")
    k = pl.kernel(
        _mf_body,
        out_type=jax.ShapeDtypeStruct((B,), jnp.float32),
        mesh=mesh,
        compiler_params=pltpu.CompilerParams(needs_layout_passes=False),
        scratch_types=[
            pltpu.VMEM((BPW,), jnp.int32),
            pltpu.VMEM((BPW,), jnp.int32),
            pltpu.VMEM((NBUF, CH, D), jnp.float32),
            pltpu.VMEM((NBUF, CH, D), jnp.float32),
            pltpu.VMEM((L * (L + 1),), jnp.float32),
            pltpu.VMEM((BPW,), jnp.float32),
            pltpu.SemaphoreType.DMA((NBUF,)),
            pltpu.SemaphoreType.DMA((NBUF,)),
        ],
    )
    return k(user_table, item_table, users_r, items_r)


def kernel(users, items, user_table, item_table):
    return _run(users, items, user_table, item_table)


# speedup vs baseline: 1.0016x; 1.0016x over previous
import jax
import jax.numpy as jnp
from jax import lax
from jax.experimental import pallas as pl
from jax.experimental.pallas import tpu as pltpu
from jax.experimental.pallas import tpu_sc as plsc

B = 16384
D = 128
NC = 2
NS = 16
L = 16
NW = NC * NS
BPW = B // NW
CH = 128
NCH = BPW // CH
NBUF = 3


def _mf_body(user_table, item_table, users_r, items_r, out_hbm,
             uidx, iidx, urows, irows, psum, out_v, sems_u, sems_i):
    wid = lax.axis_index("s") * NC + lax.axis_index("c")

    base = wid * BPW
    pltpu.sync_copy(users_r.at[pl.ds(base, BPW)], uidx)
    pltpu.sync_copy(items_r.at[pl.ds(base, BPW)], iidx)

    def start(c):
        b = c % NBUF
        sl = pl.ds(c * CH, CH)
        cu = pltpu.make_async_copy(user_table.at[uidx.at[sl]], urows.at[b],
                                   sems_u.at[b])
        ci = pltpu.make_async_copy(item_table.at[iidx.at[sl]], irows.at[b],
                                   sems_i.at[b])
        cu.start()
        ci.start()
        return cu, ci

    row_iota = lax.iota(jnp.int32, L)
    pending = [start(c) for c in range(NBUF)]
    for c in range(NCH):
        cur = pending[c % NBUF]
        cur[0].wait()
        cur[1].wait()
        b = c % NBUF
        ub = urows.at[b]
        ib = irows.at[b]

        def gbody(g, _, ub=ub, ib=ib, c=c):
            for j in range(L):
                r = g * L + j
                acc = ub[r, pl.ds(0, L)] * ib[r, pl.ds(0, L)]
                for k in range(1, D // L):
                    sl = pl.ds(k * L, L)
                    acc = acc + ub[r, sl] * ib[r, sl]
                psum[pl.ds(j * (L + 1), L)] = acc
            rows17 = row_iota * (L + 1)
            t = [plsc.load_gather(psum, [rows17 + m]) for m in range(L)]
            while len(t) > 1:
                t = [t[i] + t[i + 1] for i in range(0, len(t), 2)]
            out_v[pl.ds(c * CH + g * L, L)] = t[0]
            return 0

        lax.fori_loop(0, CH // L, gbody, 0)
        if c + NBUF < NCH:
            pending[c % NBUF] = start(c + NBUF)

    pltpu.sync_copy(out_v, out_hbm.at[pl.ds(base, BPW)])


@jax.jit
def _run(users, items, user_table, item_table):
    users_r = users.astype(jnp.int32)
    items_r = items.astype(jnp.int32)
    mesh = plsc.VectorSubcoreMesh(core_axis_name="c", subcore_axis_name="s")
    k = pl.kernel(
        _mf_body,
        out_type=jax.ShapeDtypeStruct((B,), jnp.float32),
        mesh=mesh,
        compiler_params=pltpu.CompilerParams(needs_layout_passes=False,
                                             skip_device_barrier=True),
        scratch_types=[
            pltpu.VMEM((BPW,), jnp.int32),
            pltpu.VMEM((BPW,), jnp.int32),
            pltpu.VMEM((NBUF, CH, D), jnp.float32),
            pltpu.VMEM((NBUF, CH, D), jnp.float32),
            pltpu.VMEM((L * (L + 1),), jnp.float32),
            pltpu.VMEM((BPW,), jnp.float32),
            pltpu.SemaphoreType.DMA((NBUF,)),
            pltpu.SemaphoreType.DMA((NBUF,)),
        ],
    )
    return k(user_table, item_table, users_r, items_r)


def kernel(users, items, user_table, item_table):
    return _run(users, items, user_table, item_table)
